# baseline (speedup 1.0000x reference)
import jax
import jax.numpy as jnp
from jax.experimental import pallas as pl

_TOKEN_BLOCK = 4096
_SKIP_THRESHOLD = 0.1


def _router_block(x_ref, w_ref, wout_ref, iout_ref, mout_ref):
    xb = x_ref[...]
    wg = w_ref[...]
    logits = jax.lax.dot_general(
        wg, xb, (((1,), (1,)), ((), ())),
        preferred_element_type=jnp.float32,
    )

    E = logits.shape[0]
    eidx = jax.lax.broadcasted_iota(jnp.int32, logits.shape, 0)

    m1 = jnp.max(logits, axis=0, keepdims=True)
    i1 = jnp.min(jnp.where(logits == m1, eidx, E), axis=0, keepdims=True)
    masked = jnp.where(eidx == i1, -jnp.inf, logits)
    m2 = jnp.max(masked, axis=0, keepdims=True)
    i2 = jnp.min(jnp.where(masked == m2, eidx, E), axis=0, keepdims=True)

    s = jnp.sum(jnp.exp(logits - m1), axis=0, keepdims=True)
    e2 = jnp.exp(m2 - m1)
    p1 = 1.0 / s
    denom = 1.0 + e2
    w1 = 1.0 / denom
    w2 = e2 / denom

    skip = p1 < _SKIP_THRESHOLD
    w1 = jnp.where(skip, 0.0, w1)
    w2 = jnp.where(skip, 0.0, w2)

    wout_ref[...] = jnp.concatenate([w1, w2], axis=0)
    iout_ref[...] = jnp.concatenate([i1, i2], axis=0)
    mout_ref[...] = skip


@jax.jit
def kernel(x, W_gate):
    T, H = x.shape
    E = W_gate.shape[0]
    TB = _TOKEN_BLOCK
    grid = (T // TB,)

    weights_t, idx_t, mask_t = pl.pallas_call(
        _router_block,
        grid=grid,
        in_specs=[
            pl.BlockSpec((TB, H), lambda i: (i, 0)),
            pl.BlockSpec((E, H), lambda i: (0, 0)),
        ],
        out_specs=[
            pl.BlockSpec((2, TB), lambda i: (0, i)),
            pl.BlockSpec((2, TB), lambda i: (0, i)),
            pl.BlockSpec((1, TB), lambda i: (0, i)),
        ],
        out_shape=[
            jax.ShapeDtypeStruct((2, T), jnp.float32),
            jax.ShapeDtypeStruct((2, T), jnp.int32),
            jax.ShapeDtypeStruct((1, T), jnp.bool_),
        ],
    )(x, W_gate)
    return weights_t.T, idx_t.T, mask_t.reshape(T)


# speedup vs baseline: 1.0136x; 1.0136x over previous
import jax
import jax.numpy as jnp
from jax.experimental import pallas as pl

_TOKEN_BLOCK = 4096
_SKIP_THRESHOLD = 0.1


def _router_block(x_ref, w_ref, wout_ref, iout_ref, mout_ref):
    xb = x_ref[...]
    wg = w_ref[...]
    logits = jax.lax.dot_general(
        wg, xb, (((1,), (1,)), ((), ())),
        preferred_element_type=jnp.float32,
    )

    E = logits.shape[0]
    eidx = jax.lax.broadcasted_iota(jnp.int32, logits.shape, 0)

    m1 = jnp.max(logits, axis=0, keepdims=True)
    i1 = jnp.min(jnp.where(logits == m1, eidx, E), axis=0, keepdims=True)
    masked = jnp.where(eidx == i1, -jnp.inf, logits)
    m2 = jnp.max(masked, axis=0, keepdims=True)
    i2 = jnp.min(jnp.where(masked == m2, eidx, E), axis=0, keepdims=True)

    s = jnp.sum(jnp.exp(logits - m1), axis=0, keepdims=True)
    e2 = jnp.exp(m2 - m1)
    p1 = 1.0 / s
    denom = 1.0 + e2
    w1 = 1.0 / denom
    w2 = e2 / denom

    skip = p1 < _SKIP_THRESHOLD
    w1 = jnp.where(skip, 0.0, w1)
    w2 = jnp.where(skip, 0.0, w2)

    wout_ref[...] = jnp.concatenate([w1, w2], axis=0)
    iout_ref[...] = jnp.concatenate([i1, i2], axis=0)
    mout_ref[...] = jnp.where(skip, 1.0, 0.0)


@jax.jit
def kernel(x, W_gate):
    T, H = x.shape
    E = W_gate.shape[0]
    TB = _TOKEN_BLOCK
    grid = (T // TB,)

    weights_t, idx_t, mask_t = pl.pallas_call(
        _router_block,
        grid=grid,
        in_specs=[
            pl.BlockSpec((TB, H), lambda i: (i, 0)),
            pl.BlockSpec((E, H), lambda i: (0, 0)),
        ],
        out_specs=[
            pl.BlockSpec((2, TB), lambda i: (0, i)),
            pl.BlockSpec((2, TB), lambda i: (0, i)),
            pl.BlockSpec((1, TB), lambda i: (0, i)),
        ],
        out_shape=[
            jax.ShapeDtypeStruct((2, T), jnp.float32),
            jax.ShapeDtypeStruct((2, T), jnp.int32),
            jax.ShapeDtypeStruct((1, T), jnp.float32),
        ],
    )(x, W_gate)
    return weights_t.T, idx_t.T, mask_t.reshape(T) > 0.5


# speedup vs baseline: 1.0250x; 1.0112x over previous
import jax
import jax.numpy as jnp
from jax.experimental import pallas as pl

_TOKEN_BLOCK = 4096
_SKIP_THRESHOLD = 0.1


def _router_block(x_ref, w_ref, wout_ref, iout_ref, mout_ref):
    xb = x_ref[...]
    wg = w_ref[...]
    logits = jax.lax.dot_general(
        wg, xb, (((1,), (1,)), ((), ())),
        preferred_element_type=jnp.float32,
    )

    E = logits.shape[0]
    eidx = jax.lax.broadcasted_iota(jnp.int32, logits.shape, 0)

    m1 = jnp.max(logits, axis=0, keepdims=True)
    i1 = jnp.min(jnp.where(logits == m1, eidx, E), axis=0, keepdims=True)
    masked = jnp.where(eidx == i1, -jnp.inf, logits)
    m2 = jnp.max(masked, axis=0, keepdims=True)
    i2 = jnp.min(jnp.where(masked == m2, eidx, E), axis=0, keepdims=True)

    s = jnp.sum(jnp.exp(logits - m1), axis=0, keepdims=True)
    e2 = jnp.exp(m2 - m1)
    p1 = 1.0 / s
    denom = 1.0 + e2
    w1 = 1.0 / denom
    w2 = e2 / denom

    skip = p1 < _SKIP_THRESHOLD
    w1 = jnp.where(skip, 0.0, w1)
    w2 = jnp.where(skip, 0.0, w2)

    wout_ref[...] = jnp.concatenate([w1, w2], axis=0)
    iout_ref[...] = jnp.concatenate([i1, i2], axis=0)
    mout_ref[...] = skip


@jax.jit
def kernel(x, W_gate):
    T, H = x.shape
    E = W_gate.shape[0]
    TB = _TOKEN_BLOCK
    grid = (T // TB,)

    weights_t, idx_t, mask_t = pl.pallas_call(
        _router_block,
        grid=grid,
        in_specs=[
            pl.BlockSpec((TB, H), lambda i: (i, 0)),
            pl.BlockSpec((E, H), lambda i: (0, 0)),
        ],
        out_specs=[
            pl.BlockSpec((2, TB), lambda i: (0, i)),
            pl.BlockSpec((2, TB), lambda i: (0, i)),
            pl.BlockSpec((1, TB), lambda i: (0, i)),
        ],
        out_shape=[
            jax.ShapeDtypeStruct((2, T), jnp.float32),
            jax.ShapeDtypeStruct((2, T), jnp.int32),
            jax.ShapeDtypeStruct((1, T), jnp.bool_),
        ],
    )(x, W_gate)
    return weights_t.T, idx_t.T, mask_t.reshape(T)
